# baseline (speedup 1.0000x reference)
import jax
import jax.numpy as jnp
from jax.experimental import pallas as pl


def _matmul_body(x_ref, w_ref, o_ref):
    o_ref[...] = jnp.dot(x_ref[...], w_ref[...],
                         preferred_element_type=jnp.float32)


def _mm(x, w):
    M, K = x.shape
    _, N = w.shape
    BM = 1000
    return pl.pallas_call(
        _matmul_body,
        grid=(M // BM,),
        in_specs=[pl.BlockSpec((BM, K), lambda i: (i, 0)),
                  pl.BlockSpec((K, N), lambda i: (0, 0))],
        out_specs=pl.BlockSpec((BM, N), lambda i: (i, 0)),
        out_shape=jax.ShapeDtypeStruct((M, N), jnp.float32),
    )(x, w)


def kernel(edge_index, user_emb, item_emb, W1, b1, W2, b2, W3, b3):
    n = user_emb.shape[0] + item_emb.shape[0]
    x = jnp.concatenate([user_emb, item_emb], axis=0)
    loop = jnp.arange(n, dtype=edge_index.dtype)
    src = jnp.concatenate([edge_index[0], loop])
    dst = jnp.concatenate([edge_index[1], loop])
    deg = jnp.zeros((n,), x.dtype).at[dst].add(1.0)
    dinv = jnp.where(deg > 0, 1.0 / jnp.sqrt(deg), 0.0)
    norm = dinv[src] * dinv[dst]
    for W, b in ((W1, b1), (W2, b2), (W3, b3)):
        xw = _mm(x, W)
        msgs = xw[src] * norm[:, None]
        out = jnp.zeros_like(xw).at[dst].add(msgs) + b
        x = jax.nn.relu(out)
    return x


# speedup vs baseline: 5.2776x; 5.2776x over previous
import functools

import jax
import jax.numpy as jnp
from jax import lax
from jax.experimental import pallas as pl
from jax.experimental.pallas import tpu as pltpu
from jax.experimental.pallas import tpu_sc as plsc

N_NODES = 10000
N_PAD = 10240
D = 512
NC = 2
NS = 16
GARB = N_NODES
EB = 128
CW = 128
NCH = D // CW
ROWS_T = N_PAD // NS


def _sc_mesh():
    return plsc.VectorSubcoreMesh(core_axis_name="c", subcore_axis_name="s")


def _deg_body(dst_ref, deg_ref, hist, didx, ones_s, zbuf):
    cid = lax.axis_index("c")
    sid = lax.axis_index("s")
    rpt = didx.shape[0]

    def fz(i, _):
        zbuf[i] = jnp.zeros((16,), jnp.float32)
        return 0
    lax.fori_loop(0, ROWS_T, fz, 0)

    def fo(i, _):
        ones_s[i] = jnp.ones((16,), jnp.float32)
        return 0
    lax.fori_loop(0, EB, fo, 0)

    pltpu.sync_copy(zbuf, hist.at[pl.ds(sid * ROWS_T, ROWS_T)])
    plsc.subcore_barrier()

    base = (cid * NS + sid) * rpt
    pltpu.sync_copy(dst_ref.at[pl.ds(base, rpt)], didx)

    def fe(j, _):
        pltpu.sync_copy(ones_s, hist.at[didx.at[j]], add=True)
        return 0
    lax.fori_loop(0, rpt, fe, 0)

    plsc.subcore_barrier()
    pltpu.sync_copy(hist.at[pl.ds(sid * ROWS_T, ROWS_T)],
                    deg_ref.at[cid, pl.ds(sid * ROWS_T, ROWS_T)])


def _deg(dst_p):
    nrows = dst_p.shape[0]
    rpt = nrows // (NC * NS)
    f = pl.kernel(
        _deg_body,
        out_type=jax.ShapeDtypeStruct((NC, N_PAD, 16), jnp.float32),
        mesh=_sc_mesh(),
        scratch_types=[
            pltpu.VMEM_SHARED((N_PAD, 16), jnp.float32),
            pltpu.VMEM((rpt, EB), jnp.int32),
            pltpu.VMEM((EB, 16), jnp.float32),
            pltpu.VMEM((ROWS_T, 16), jnp.float32),
        ],
    )
    return f(dst_p)


def _agg_body(t0, t1, t2, t3, src_ref, dst_ref,
              o0, o1, o2, o3, acc, sidx, didx, gbuf, sem):
    cid = lax.axis_index("c")
    sid = lax.axis_index("s")
    rpt = sidx.shape[0]
    base = sid * rpt
    pltpu.sync_copy(src_ref.at[pl.ds(base, rpt)], sidx)
    pltpu.sync_copy(dst_ref.at[pl.ds(base, rpt)], didx)
    tins = (t0, t1, t2, t3)
    touts = (o0, o1, o2, o3)
    for c in range(NC):
        @pl.when(cid == c)
        def _(c=c):
            for k in range(NCH // NC):
                ch = (NCH // NC) * c + k
                t_ref = tins[ch]
                o_ref = touts[ch]
                pltpu.sync_copy(t_ref.at[pl.ds(sid * ROWS_T, ROWS_T)],
                                acc.at[pl.ds(sid * ROWS_T, ROWS_T)])
                plsc.subcore_barrier()

                def fe(j, _):
                    pltpu.async_copy(t_ref.at[sidx.at[j]], gbuf, sem).wait()
                    pltpu.sync_copy(gbuf, acc.at[didx.at[j]], add=True)
                    return 0
                lax.fori_loop(0, rpt, fe, 0)

                plsc.subcore_barrier()
                pltpu.sync_copy(acc.at[pl.ds(sid * ROWS_T, ROWS_T)],
                                o_ref.at[pl.ds(sid * ROWS_T, ROWS_T)])
                plsc.subcore_barrier()


def _agg(t4, src_p, dst_p):
    nrows = src_p.shape[0]
    rpt = nrows // NS
    f = pl.kernel(
        _agg_body,
        out_type=tuple(jax.ShapeDtypeStruct((N_PAD, CW), jnp.float32)
                       for _ in range(NCH)),
        mesh=_sc_mesh(),
        scratch_types=[
            pltpu.VMEM_SHARED((N_PAD, CW), jnp.float32),
            pltpu.VMEM((rpt, EB), jnp.int32),
            pltpu.VMEM((rpt, EB), jnp.int32),
            pltpu.VMEM((EB, CW), jnp.float32),
            pltpu.SemaphoreType.DMA,
        ],
    )
    return f(*t4, src_p, dst_p)


_BM = 1024


def _l1_body(x_ref, w_ref, d_ref, o0, o1, o2, o3):
    dinv = lax.rsqrt(d_ref[...] + 1.0)
    t = jnp.dot(x_ref[...], w_ref[...],
                preferred_element_type=jnp.float32) * dinv
    for c, o in enumerate((o0, o1, o2, o3)):
        o[...] = t[:, c * CW:(c + 1) * CW]


def _tc_l1(x, W, deg1):
    return pl.pallas_call(
        _l1_body,
        grid=(N_PAD // _BM,),
        in_specs=[pl.BlockSpec((_BM, D), lambda i: (i, 0)),
                  pl.BlockSpec((D, D), lambda i: (0, 0)),
                  pl.BlockSpec((_BM, 1), lambda i: (i, 0))],
        out_specs=[pl.BlockSpec((_BM, CW), lambda i: (i, 0))] * NCH,
        out_shape=[jax.ShapeDtypeStruct((N_PAD, CW), jnp.float32)] * NCH,
    )(x, W, deg1)


def _mid_body(s0, s1, s2, s3, d_ref, b_ref, w_ref, o0, o1, o2, o3):
    dinv = lax.rsqrt(d_ref[...] + 1.0)
    s = jnp.concatenate([s0[...], s1[...], s2[...], s3[...]], axis=1)
    x = jnp.maximum(s * dinv + b_ref[...], 0.0)
    t = jnp.dot(x, w_ref[...], preferred_element_type=jnp.float32) * dinv
    for c, o in enumerate((o0, o1, o2, o3)):
        o[...] = t[:, c * CW:(c + 1) * CW]


def _tc_mid(s4, deg1, b_prev, W):
    return pl.pallas_call(
        _mid_body,
        grid=(N_PAD // _BM,),
        in_specs=[pl.BlockSpec((_BM, CW), lambda i: (i, 0))] * NCH
        + [pl.BlockSpec((_BM, 1), lambda i: (i, 0)),
           pl.BlockSpec((1, D), lambda i: (0, 0)),
           pl.BlockSpec((D, D), lambda i: (0, 0))],
        out_specs=[pl.BlockSpec((_BM, CW), lambda i: (i, 0))] * NCH,
        out_shape=[jax.ShapeDtypeStruct((N_PAD, CW), jnp.float32)] * NCH,
    )(*s4, deg1, b_prev, W)


def _out_body(s0, s1, s2, s3, d_ref, b_ref, o_ref):
    dinv = lax.rsqrt(d_ref[...] + 1.0)
    s = jnp.concatenate([s0[...], s1[...], s2[...], s3[...]], axis=1)
    o_ref[...] = jnp.maximum(s * dinv + b_ref[...], 0.0)


def _tc_out(s4, deg1, b_last):
    return pl.pallas_call(
        _out_body,
        grid=(N_PAD // _BM,),
        in_specs=[pl.BlockSpec((_BM, CW), lambda i: (i, 0))] * NCH
        + [pl.BlockSpec((_BM, 1), lambda i: (i, 0)),
           pl.BlockSpec((1, D), lambda i: (0, 0))],
        out_specs=pl.BlockSpec((_BM, D), lambda i: (i, 0)),
        out_shape=jax.ShapeDtypeStruct((N_PAD, D), jnp.float32),
    )(*s4, deg1, b_last)


def kernel(edge_index, user_emb, item_emb, W1, b1, W2, b2, W3, b3):
    E = edge_index.shape[1]
    nrows = -(-E // (EB * NC * NS)) * NC * NS
    ep = nrows * EB
    src_p = jnp.concatenate(
        [edge_index[0], jnp.zeros((ep - E,), jnp.int32)]).reshape(nrows, EB)
    dst_p = jnp.concatenate(
        [edge_index[1], jnp.full((ep - E,), GARB, jnp.int32)]).reshape(nrows, EB)

    x0 = jnp.concatenate(
        [user_emb, item_emb,
         jnp.zeros((N_PAD - N_NODES, D), jnp.float32)], axis=0)

    degp = _deg(dst_p)
    deg1 = degp[0, :, :1] + degp[1, :, :1]

    t = _tc_l1(x0, W1, deg1)
    s = _agg(t, src_p, dst_p)
    t = _tc_mid(s, deg1, b1.reshape(1, D), W2)
    s = _agg(t, src_p, dst_p)
    t = _tc_mid(s, deg1, b2.reshape(1, D), W3)
    s = _agg(t, src_p, dst_p)
    return _tc_out(s, deg1, b3.reshape(1, D))[:N_NODES]


# speedup vs baseline: 6.4414x; 1.2205x over previous
import functools

import jax
import jax.numpy as jnp
from jax import lax
from jax.experimental import pallas as pl
from jax.experimental.pallas import tpu as pltpu
from jax.experimental.pallas import tpu_sc as plsc

N_NODES = 10000
N_PAD = 10240
D = 512
NC = 2
NS = 16
GARB = N_NODES
EB = 128
CW = 128
NCH = D // CW
ROWS_T = N_PAD // NS


def _sc_mesh():
    return plsc.VectorSubcoreMesh(core_axis_name="c", subcore_axis_name="s")


def _deg_body(dst_ref, deg_ref, hist, didx, ones_s, zbuf):
    cid = lax.axis_index("c")
    sid = lax.axis_index("s")
    rpt = didx.shape[0]

    def fz(i, _):
        zbuf[i] = jnp.zeros((16,), jnp.float32)
        return 0
    lax.fori_loop(0, ROWS_T, fz, 0)

    def fo(i, _):
        ones_s[i] = jnp.ones((16,), jnp.float32)
        return 0
    lax.fori_loop(0, EB, fo, 0)

    pltpu.sync_copy(zbuf, hist.at[pl.ds(sid * ROWS_T, ROWS_T)])
    plsc.subcore_barrier()

    base = (cid * NS + sid) * rpt
    pltpu.sync_copy(dst_ref.at[pl.ds(base, rpt)], didx)

    def fe(j, _):
        pltpu.sync_copy(ones_s, hist.at[didx.at[j]], add=True)
        return 0
    lax.fori_loop(0, rpt, fe, 0)

    plsc.subcore_barrier()
    pltpu.sync_copy(hist.at[pl.ds(sid * ROWS_T, ROWS_T)],
                    deg_ref.at[cid, pl.ds(sid * ROWS_T, ROWS_T)])


def _deg(dst_p):
    nrows = dst_p.shape[0]
    rpt = nrows // (NC * NS)
    f = pl.kernel(
        _deg_body,
        out_type=jax.ShapeDtypeStruct((NC, N_PAD, 16), jnp.float32),
        mesh=_sc_mesh(),
        scratch_types=[
            pltpu.VMEM_SHARED((N_PAD, 16), jnp.float32),
            pltpu.VMEM((rpt, EB), jnp.int32),
            pltpu.VMEM((EB, 16), jnp.float32),
            pltpu.VMEM((ROWS_T, 16), jnp.float32),
        ],
    )
    return f(dst_p)


def _edge_loop(t_ref, acc, sidx, didx, gb0, gb1, gs0, gs1, ss0, ss1):
    rpt = sidx.shape[0]

    def gstart(j, gb, gs):
        pltpu.make_async_copy(t_ref.at[sidx.at[j]], gb, gs).start()

    def gwait(j, gb, gs):
        pltpu.make_async_copy(t_ref.at[sidx.at[j]], gb, gs).wait()

    def sadd(j, gb, ss):
        pltpu.make_async_copy(gb, acc.at[didx.at[j]], ss).start(add=True)
        pltpu.make_async_copy(gb, acc.at[didx.at[j]], ss).wait()

    gstart(0, gb0, gs0)
    gstart(1, gb1, gs1)

    def pair(p, _):
        j0 = 2 * p
        j1 = j0 + 1
        gwait(j0, gb0, gs0)
        sadd(j0, gb0, ss0)
        gstart(j0 + 2, gb0, gs0)
        gwait(j1, gb1, gs1)
        sadd(j1, gb1, ss1)
        gstart(j1 + 2, gb1, gs1)
        return 0

    lax.fori_loop(0, rpt // 2 - 1, pair, 0)
    gwait(rpt - 2, gb0, gs0)
    sadd(rpt - 2, gb0, ss0)
    gwait(rpt - 1, gb1, gs1)
    sadd(rpt - 1, gb1, ss1)


def _agg_body(t0, t1, t2, t3, src_ref, dst_ref,
              o0, o1, o2, o3, acc, sidx, didx, gb0, gb1,
              gs0, gs1, ss0, ss1):
    cid = lax.axis_index("c")
    sid = lax.axis_index("s")
    hrows = sidx.shape[0]
    base = sid * (2 * hrows)
    tins = (t0, t1, t2, t3)
    touts = (o0, o1, o2, o3)
    for k in range(NCH // NC):
        for c in range(NC):
            @pl.when(cid == c)
            def _(c=c, k=k):
                ch = (NCH // NC) * c + k
                pltpu.sync_copy(tins[ch].at[pl.ds(sid * ROWS_T, ROWS_T)],
                                acc.at[pl.ds(sid * ROWS_T, ROWS_T)])
                plsc.subcore_barrier()
        for h in range(2):
            pltpu.sync_copy(src_ref.at[pl.ds(base + h * hrows, hrows)], sidx)
            pltpu.sync_copy(dst_ref.at[pl.ds(base + h * hrows, hrows)], didx)
            for c in range(NC):
                @pl.when(cid == c)
                def _(c=c, k=k):
                    ch = (NCH // NC) * c + k
                    _edge_loop(tins[ch], acc, sidx, didx,
                               gb0, gb1, gs0, gs1, ss0, ss1)
        for c in range(NC):
            @pl.when(cid == c)
            def _(c=c, k=k):
                ch = (NCH // NC) * c + k
                plsc.subcore_barrier()
                pltpu.sync_copy(acc.at[pl.ds(sid * ROWS_T, ROWS_T)],
                                touts[ch].at[pl.ds(sid * ROWS_T, ROWS_T)])
                plsc.subcore_barrier()


def _agg(t4, src_p, dst_p):
    nrows = src_p.shape[0]
    rpt = nrows // NS // 2
    f = pl.kernel(
        _agg_body,
        out_type=tuple(jax.ShapeDtypeStruct((N_PAD, CW), jnp.float32)
                       for _ in range(NCH)),
        mesh=_sc_mesh(),
        scratch_types=[
            pltpu.VMEM_SHARED((N_PAD, CW), jnp.float32),
            pltpu.VMEM((rpt, EB), jnp.int32),
            pltpu.VMEM((rpt, EB), jnp.int32),
            pltpu.VMEM((EB, CW), jnp.float32),
            pltpu.VMEM((EB, CW), jnp.float32),
            pltpu.SemaphoreType.DMA,
            pltpu.SemaphoreType.DMA,
            pltpu.SemaphoreType.DMA,
            pltpu.SemaphoreType.DMA,
        ],
    )
    return f(*t4, src_p, dst_p)


_BM = 1024


def _l1_body(x_ref, w_ref, d_ref, o0, o1, o2, o3):
    dinv = lax.rsqrt(d_ref[...] + 1.0)
    t = jnp.dot(x_ref[...], w_ref[...],
                preferred_element_type=jnp.float32) * dinv
    for c, o in enumerate((o0, o1, o2, o3)):
        o[...] = t[:, c * CW:(c + 1) * CW]


def _tc_l1(x, W, deg1):
    return pl.pallas_call(
        _l1_body,
        grid=(N_PAD // _BM,),
        in_specs=[pl.BlockSpec((_BM, D), lambda i: (i, 0)),
                  pl.BlockSpec((D, D), lambda i: (0, 0)),
                  pl.BlockSpec((_BM, 1), lambda i: (i, 0))],
        out_specs=[pl.BlockSpec((_BM, CW), lambda i: (i, 0))] * NCH,
        out_shape=[jax.ShapeDtypeStruct((N_PAD, CW), jnp.float32)] * NCH,
    )(x, W, deg1)


def _mid_body(s0, s1, s2, s3, d_ref, b_ref, w_ref, o0, o1, o2, o3):
    dinv = lax.rsqrt(d_ref[...] + 1.0)
    s = jnp.concatenate([s0[...], s1[...], s2[...], s3[...]], axis=1)
    x = jnp.maximum(s * dinv + b_ref[...], 0.0)
    t = jnp.dot(x, w_ref[...], preferred_element_type=jnp.float32) * dinv
    for c, o in enumerate((o0, o1, o2, o3)):
        o[...] = t[:, c * CW:(c + 1) * CW]


def _tc_mid(s4, deg1, b_prev, W):
    return pl.pallas_call(
        _mid_body,
        grid=(N_PAD // _BM,),
        in_specs=[pl.BlockSpec((_BM, CW), lambda i: (i, 0))] * NCH
        + [pl.BlockSpec((_BM, 1), lambda i: (i, 0)),
           pl.BlockSpec((1, D), lambda i: (0, 0)),
           pl.BlockSpec((D, D), lambda i: (0, 0))],
        out_specs=[pl.BlockSpec((_BM, CW), lambda i: (i, 0))] * NCH,
        out_shape=[jax.ShapeDtypeStruct((N_PAD, CW), jnp.float32)] * NCH,
    )(*s4, deg1, b_prev, W)


def _out_body(s0, s1, s2, s3, d_ref, b_ref, o_ref):
    dinv = lax.rsqrt(d_ref[...] + 1.0)
    s = jnp.concatenate([s0[...], s1[...], s2[...], s3[...]], axis=1)
    o_ref[...] = jnp.maximum(s * dinv + b_ref[...], 0.0)


def _tc_out(s4, deg1, b_last):
    return pl.pallas_call(
        _out_body,
        grid=(N_PAD // _BM,),
        in_specs=[pl.BlockSpec((_BM, CW), lambda i: (i, 0))] * NCH
        + [pl.BlockSpec((_BM, 1), lambda i: (i, 0)),
           pl.BlockSpec((1, D), lambda i: (0, 0))],
        out_specs=pl.BlockSpec((_BM, D), lambda i: (i, 0)),
        out_shape=jax.ShapeDtypeStruct((N_PAD, D), jnp.float32),
    )(*s4, deg1, b_last)


def kernel(edge_index, user_emb, item_emb, W1, b1, W2, b2, W3, b3):
    E = edge_index.shape[1]
    align = NC * NS * 8
    nrows = -(-E // (EB * align)) * align
    ep = nrows * EB
    src_p = jnp.concatenate(
        [edge_index[0], jnp.zeros((ep - E,), jnp.int32)]).reshape(nrows, EB)
    dst_p = jnp.concatenate(
        [edge_index[1], jnp.full((ep - E,), GARB, jnp.int32)]).reshape(nrows, EB)

    x0 = jnp.concatenate(
        [user_emb, item_emb,
         jnp.zeros((N_PAD - N_NODES, D), jnp.float32)], axis=0)

    degp = _deg(dst_p)
    deg1 = degp[0, :, :1] + degp[1, :, :1]

    t = _tc_l1(x0, W1, deg1)
    s = _agg(t, src_p, dst_p)
    t = _tc_mid(s, deg1, b1.reshape(1, D), W2)
    s = _agg(t, src_p, dst_p)
    t = _tc_mid(s, deg1, b2.reshape(1, D), W3)
    s = _agg(t, src_p, dst_p)
    return _tc_out(s, deg1, b3.reshape(1, D))[:N_NODES]
